# baseline (speedup 1.0000x reference)
import functools

import jax
import jax.numpy as jnp
from jax.experimental import pallas as pl

_NUM_REPS = 512
_CODE_DIM = 32
_BLOCK_N = 2048


def _cluster_block_kernel(emb_ref, cen_ref, rep_ref, loss_ref):
    i = pl.program_id(0)
    emb = emb_ref[:]
    cen = cen_ref[:]
    csq = jnp.sum(cen * cen, axis=1, keepdims=True)
    cross = jax.lax.dot_general(
        cen, -2.0 * emb, (((1,), (1,)), ((), ())),
        preferred_element_type=jnp.float32)
    t = cross + csq
    md = jnp.min(t, axis=0, keepdims=True)
    k_iota = jax.lax.broadcasted_iota(jnp.int32, t.shape, 0)
    rep = jnp.min(jnp.where(t == md, k_iota, _NUM_REPS), axis=0)
    rep_ref[:] = rep[None, None, :]

    part = jnp.sum(emb * emb) + jnp.sum(md)

    @pl.when(i == 0)
    def _init():
        loss_ref[:, :] = jnp.zeros((1, 1), jnp.float32)

    loss_ref[:, :] += part.reshape(1, 1)


@functools.partial(jax.jit, static_argnums=())
def _cluster(embs, centers):
    n = embs.shape[0]
    grid = (n // _BLOCK_N,)
    rep3d, loss = pl.pallas_call(
        _cluster_block_kernel,
        grid=grid,
        in_specs=[
            pl.BlockSpec((_BLOCK_N, _CODE_DIM), lambda i: (i, 0)),
            pl.BlockSpec((_NUM_REPS, _CODE_DIM), lambda i: (0, 0)),
        ],
        out_specs=[
            pl.BlockSpec((1, 1, _BLOCK_N), lambda i: (i, 0, 0)),
            pl.BlockSpec((1, 1), lambda i: (0, 0)),
        ],
        out_shape=[
            jax.ShapeDtypeStruct((n // _BLOCK_N, 1, _BLOCK_N), jnp.int32),
            jax.ShapeDtypeStruct((1, 1), jnp.float32),
        ],
    )(embs, centers)
    return rep3d, loss


def kernel(embs, centers):
    rep3d, loss = _cluster(embs, centers)
    return (centers, rep3d.reshape(embs.shape[0]), loss[0, 0])


# speedup vs baseline: 1.0400x; 1.0400x over previous
import functools

import jax
import jax.numpy as jnp
from jax.experimental import pallas as pl

_NUM_REPS = 512
_CODE_DIM = 32
_BLOCK_N = 4096


def _cluster_block_kernel(emb_ref, cen_ref, rep_ref, loss_ref):
    i = pl.program_id(0)
    emb = emb_ref[:]
    cen = cen_ref[:]
    csq = jnp.sum(cen * cen, axis=1, keepdims=True)
    cross = jax.lax.dot_general(
        cen, -2.0 * emb, (((1,), (1,)), ((), ())),
        preferred_element_type=jnp.float32)
    t = cross + csq
    md = jnp.min(t, axis=0, keepdims=True)
    k_iota = jax.lax.broadcasted_iota(jnp.int32, t.shape, 0)
    rep = jnp.min(jnp.where(t == md, k_iota, _NUM_REPS), axis=0)
    rep_ref[:] = rep[None, None, :]

    part = jnp.sum(emb * emb) + jnp.sum(md)

    @pl.when(i == 0)
    def _init():
        loss_ref[:, :] = jnp.zeros((1, 1), jnp.float32)

    loss_ref[:, :] += part.reshape(1, 1)


@functools.partial(jax.jit, static_argnums=())
def _cluster(embs, centers):
    n = embs.shape[0]
    grid = (n // _BLOCK_N,)
    rep3d, loss = pl.pallas_call(
        _cluster_block_kernel,
        grid=grid,
        in_specs=[
            pl.BlockSpec((_BLOCK_N, _CODE_DIM), lambda i: (i, 0)),
            pl.BlockSpec((_NUM_REPS, _CODE_DIM), lambda i: (0, 0)),
        ],
        out_specs=[
            pl.BlockSpec((1, 1, _BLOCK_N), lambda i: (i, 0, 0)),
            pl.BlockSpec((1, 1), lambda i: (0, 0)),
        ],
        out_shape=[
            jax.ShapeDtypeStruct((n // _BLOCK_N, 1, _BLOCK_N), jnp.int32),
            jax.ShapeDtypeStruct((1, 1), jnp.float32),
        ],
    )(embs, centers)
    return rep3d, loss


def kernel(embs, centers):
    rep3d, loss = _cluster(embs, centers)
    return (centers, rep3d.reshape(embs.shape[0]), loss[0, 0])
